# Initial kernel scaffold; baseline (speedup 1.0000x reference)
import jax
import jax.numpy as jnp
from jax.experimental import pallas as pl


def kernel(x, W1, b1, g1, be1, W2, b2, g2, be2, W3, b3, g3, be3, W4, b4, g4, be4, fw1, fb1, fw2, fb2, pw1, pb1, pw2, pb2):
    raise NotImplementedError("write your pallas kernel here")


# speedup vs baseline: 1.5213x; 1.5213x over previous
import functools
import jax
import jax.numpy as jnp
from jax import lax
from jax.experimental import pallas as pl
from jax.experimental.pallas import tpu as pltpu

K = 20
N = 1024
B = 2
NEG = -3.0e38


def _leaky(x):
    return jnp.where(x >= 0, x, 0.01 * x)


def _bdot(a, b):
    return jax.lax.dot_general(a.astype(jnp.bfloat16), b.astype(jnp.bfloat16),
                               (((1,), (0,)), ((), ())),
                               preferred_element_type=jnp.float32)


def _knn_body(x_ref, idx_ref):
    xb = x_ref[0]
    xb16 = xb.astype(jnp.bfloat16)
    G = jax.lax.dot_general(xb16, xb16, (((1,), (1,)), ((), ())),
                            preferred_element_type=jnp.float32)
    xx = jnp.sum(xb * xb, axis=1, keepdims=True)
    D = (-xx - (-2.0 * G)) - xx.reshape(1, N)
    col = jax.lax.broadcasted_iota(jnp.int32, (N, N), 1)
    for t in range(K):
        m = jnp.max(D, axis=1, keepdims=True)
        c = jnp.min(jnp.where(D >= m, col, N), axis=1)
        idx_ref[0, t, :] = c
        D = jnp.where(col == c[:, None], NEG, D)


def _knn(xt):
    b, n, C = xt.shape
    return pl.pallas_call(
        _knn_body,
        grid=(b,),
        in_specs=[pl.BlockSpec((1, N, C), lambda i: (i, 0, 0))],
        out_specs=pl.BlockSpec((1, K, N), lambda i: (i, 0, 0)),
        out_shape=jax.ShapeDtypeStruct((b, K, N), jnp.int32),
    )(xt)


def _conv_body(e_ref, x_ref, wa_ref, wb_ref, b_ref, m_ref, t1_ref, t2_ref):
    blkn = x_ref.shape[1]
    Cp = x_ref.shape[2]
    O = wa_ref.shape[1]
    e = e_ref[0].reshape(K * blkn, Cp)
    y1 = _bdot(e, wa_ref[...])
    y2 = _bdot(x_ref[0], wb_ref[...]) + b_ref[...]
    y = y1.reshape(K, blkn, O) + y2[None, :, :]
    m_ref[0] = jnp.max(y, axis=0)
    t1p = jnp.sum(y, axis=(0, 1), keepdims=False).reshape(1, O)
    t2p = jnp.sum(y * y, axis=(0, 1), keepdims=False).reshape(1, O)

    @pl.when((pl.program_id(0) == 0) & (pl.program_id(1) == 0))
    def _():
        t1_ref[...] = jnp.zeros_like(t1_ref)
        t2_ref[...] = jnp.zeros_like(t2_ref)

    t1_ref[...] += t1p
    t2_ref[...] += t2p


def _conv_reduce(E3, xt, waT, wbT, bias, blkn=128):
    b, n, Cp = xt.shape
    O = waT.shape[1]
    nb = n // blkn
    return pl.pallas_call(
        _conv_body,
        grid=(b, nb),
        in_specs=[
            pl.BlockSpec((1, K, blkn, Cp), lambda i, j: (i, 0, j, 0)),
            pl.BlockSpec((1, blkn, Cp), lambda i, j: (i, j, 0)),
            pl.BlockSpec((Cp, O), lambda i, j: (0, 0)),
            pl.BlockSpec((Cp, O), lambda i, j: (0, 0)),
            pl.BlockSpec((1, O), lambda i, j: (0, 0)),
        ],
        out_specs=[
            pl.BlockSpec((1, blkn, O), lambda i, j: (i, j, 0)),
            pl.BlockSpec((1, O), lambda i, j: (0, 0)),
            pl.BlockSpec((1, O), lambda i, j: (0, 0)),
        ],
        out_shape=[
            jax.ShapeDtypeStruct((b, n, O), jnp.float32),
            jax.ShapeDtypeStruct((1, O), jnp.float32),
            jax.ShapeDtypeStruct((1, O), jnp.float32),
        ],
    )(E3, xt, waT, wbT, bias.reshape(1, O))


def _bn_body(m_ref, t1_ref, t2_ref, h_ref):
    cnt = float(B * N * K)
    mean = t1_ref[...] / cnt
    var = t2_ref[...] / cnt - mean * mean
    h = (m_ref[...] - mean) / jnp.sqrt(var + 1e-5)
    h_ref[...] = _leaky(h)


def _bn_act(M, T1, T2):
    b, n, O = M.shape
    return pl.pallas_call(
        _bn_body,
        out_shape=jax.ShapeDtypeStruct((b * n, O), jnp.float32),
    )(M.reshape(b * n, O), T1, T2)


def _head_body(h_ref, fw1_ref, fb1_ref, fw2_ref, fb2_ref, pw1_ref, pb1_ref,
               pw2_ref, pb2_ref, o_ref):
    h = h_ref[...]
    parts = []
    for i in range(B):
        hb = h[i * N:(i + 1) * N]
        parts.append(jnp.concatenate(
            [jnp.mean(hb, axis=0, keepdims=True),
             jnp.max(hb, axis=0, keepdims=True)], axis=1))
    z = jnp.concatenate(parts, axis=0)
    e = _leaky(_bdot(z, fw1_ref[...]) + fb1_ref[...])
    e = _bdot(e, fw2_ref[...]) + fb2_ref[...]
    p = _leaky(_bdot(e, pw1_ref[...]) + pb1_ref[...])
    o_ref[...] = _bdot(p, pw2_ref[...]) + pb2_ref[...]


def _head(h_flat, fw1, fb1, fw2, fb2, pw1, pb1, pw2, pb2):
    return pl.pallas_call(
        _head_body,
        out_shape=jax.ShapeDtypeStruct((B, 128), jnp.float32),
    )(h_flat, fw1.T, fb1.reshape(1, -1), fw2.T, fb2.reshape(1, -1),
      pw1.T, pb1.reshape(1, -1), pw2.T, pb2.reshape(1, -1))


def _edge_features(xt, idx):
    feat = jax.vmap(lambda xb, ib: xb[ib])(xt, idx)
    return feat - xt[:, None, :, :]


def _layer(xt, W, b):
    bb, n, C = xt.shape
    O = W.shape[0]
    Cp = xt.shape[2]
    idx = _knn(xt)
    E3 = _edge_features(xt, idx)
    waT = W[:, :C].T
    wbT = W[:, C:].T
    M, T1, T2 = _conv_reduce(E3, xt, waT, wbT, b)
    h = _bn_act(M, T1, T2)
    return h.reshape(bb, n, O)


@jax.jit
def kernel(x, W1, b1, g1, be1, W2, b2, g2, be2, W3, b3, g3, be3, W4, b4, g4,
           be4, fw1, fb1, fw2, fb2, pw1, pb1, pw2, pb2):
    xt = jnp.transpose(x, (0, 2, 1))
    xt = jnp.concatenate(
        [xt, jnp.zeros((B, N, 13), jnp.float32)], axis=2)

    h = _layer1(xt, W1, b1)
    h = _layer(h, W2, b2)
    h = _layer(h, W3, b3)
    h = _layer(h, W4, b4)
    return _head(h.reshape(B * N, -1), fw1, fb1, fw2, fb2, pw1, pb1, pw2, pb2)


def _layer1(xtp, W, b):
    bb, n, Cp = xtp.shape
    C = 3
    O = W.shape[0]
    idx = _knn(xtp)
    E3 = _edge_features(xtp, idx)
    z = jnp.zeros((Cp - C, O), jnp.float32)
    waT = jnp.concatenate([W[:, :C].T, z], axis=0)
    wbT = jnp.concatenate([W[:, C:].T, z], axis=0)
    M, T1, T2 = _conv_reduce(E3, xtp, waT, wbT, b)
    h = _bn_act(M, T1, T2)
    return h.reshape(bb, n, O)


# speedup vs baseline: 5.5252x; 3.6319x over previous
import functools
import jax
import jax.numpy as jnp
from jax import lax
from jax.experimental import pallas as pl
from jax.experimental.pallas import tpu as pltpu
from jax.experimental.pallas import tpu_sc as plsc

K = 20
N = 1024
B = 2
NEG = -3.0e38


def _leaky(x):
    return jnp.where(x >= 0, x, 0.01 * x)


def _bdot(a, b):
    return jax.lax.dot_general(a.astype(jnp.bfloat16), b.astype(jnp.bfloat16),
                               (((1,), (0,)), ((), ())),
                               preferred_element_type=jnp.float32)


def _knn_body(x_ref, idx_ref):
    xb = x_ref[0]
    xb16 = xb.astype(jnp.bfloat16)
    G = jax.lax.dot_general(xb16, xb16, (((1,), (1,)), ((), ())),
                            preferred_element_type=jnp.float32)
    xx = jnp.sum(xb * xb, axis=1, keepdims=True)
    D = (-xx - (-2.0 * G)) - xx.reshape(1, N)
    col = jax.lax.broadcasted_iota(jnp.int32, (N, N), 1)
    for t in range(K):
        m = jnp.max(D, axis=1, keepdims=True)
        c = jnp.min(jnp.where(D >= m, col, N), axis=1)
        idx_ref[0, t, :] = c
        D = jnp.where(col == c[:, None], NEG, D)


def _knn(xt):
    b, n, C = xt.shape
    return pl.pallas_call(
        _knn_body,
        grid=(b,),
        in_specs=[pl.BlockSpec((1, N, C), lambda i: (i, 0, 0))],
        out_specs=pl.BlockSpec((1, K, N), lambda i: (i, 0, 0)),
        out_shape=jax.ShapeDtypeStruct((b, K, N), jnp.int32),
    )(xt)


def _conv_body(e_ref, x_ref, wa_ref, wb_ref, b_ref, m_ref, t1_ref, t2_ref):
    blkn = x_ref.shape[1]
    Cp = x_ref.shape[2]
    O = wa_ref.shape[1]
    e = e_ref[0].reshape(K * blkn, Cp)
    y1 = _bdot(e, wa_ref[...])
    y2 = _bdot(x_ref[0], wb_ref[...]) + b_ref[...]
    y = y1.reshape(K, blkn, O) + y2[None, :, :]
    m_ref[0] = jnp.max(y, axis=0)
    t1p = jnp.sum(y, axis=(0, 1), keepdims=False).reshape(1, O)
    t2p = jnp.sum(y * y, axis=(0, 1), keepdims=False).reshape(1, O)

    @pl.when((pl.program_id(0) == 0) & (pl.program_id(1) == 0))
    def _():
        t1_ref[...] = jnp.zeros_like(t1_ref)
        t2_ref[...] = jnp.zeros_like(t2_ref)

    t1_ref[...] += t1p
    t2_ref[...] += t2p


def _conv_reduce(E3, xt, waT, wbT, bias, blkn=128):
    b, n, Cp = xt.shape
    O = waT.shape[1]
    nb = n // blkn
    return pl.pallas_call(
        _conv_body,
        grid=(b, nb),
        in_specs=[
            pl.BlockSpec((1, K, blkn, Cp), lambda i, j: (i, 0, j, 0)),
            pl.BlockSpec((1, blkn, Cp), lambda i, j: (i, j, 0)),
            pl.BlockSpec((Cp, O), lambda i, j: (0, 0)),
            pl.BlockSpec((Cp, O), lambda i, j: (0, 0)),
            pl.BlockSpec((1, O), lambda i, j: (0, 0)),
        ],
        out_specs=[
            pl.BlockSpec((1, blkn, O), lambda i, j: (i, j, 0)),
            pl.BlockSpec((1, O), lambda i, j: (0, 0)),
            pl.BlockSpec((1, O), lambda i, j: (0, 0)),
        ],
        out_shape=[
            jax.ShapeDtypeStruct((b, n, O), jnp.float32),
            jax.ShapeDtypeStruct((1, O), jnp.float32),
            jax.ShapeDtypeStruct((1, O), jnp.float32),
        ],
    )(E3, xt, waT, wbT, bias.reshape(1, O))


def _bn_body(m_ref, t1_ref, t2_ref, h_ref):
    cnt = float(B * N * K)
    mean = t1_ref[...] / cnt
    var = t2_ref[...] / cnt - mean * mean
    h = (m_ref[...] - mean) / jnp.sqrt(var + 1e-5)
    h_ref[...] = _leaky(h)


def _bn_act(M, T1, T2):
    b, n, O = M.shape
    return pl.pallas_call(
        _bn_body,
        out_shape=jax.ShapeDtypeStruct((b * n, O), jnp.float32),
    )(M.reshape(b * n, O), T1, T2)


def _head_body(h_ref, fw1_ref, fb1_ref, fw2_ref, fb2_ref, pw1_ref, pb1_ref,
               pw2_ref, pb2_ref, o_ref):
    h = h_ref[...]
    parts = []
    for i in range(B):
        hb = h[i * N:(i + 1) * N]
        parts.append(jnp.concatenate(
            [jnp.mean(hb, axis=0, keepdims=True),
             jnp.max(hb, axis=0, keepdims=True)], axis=1))
    z = jnp.concatenate(parts, axis=0)
    e = _leaky(_bdot(z, fw1_ref[...]) + fb1_ref[...])
    e = _bdot(e, fw2_ref[...]) + fb2_ref[...]
    p = _leaky(_bdot(e, pw1_ref[...]) + pb1_ref[...])
    o_ref[...] = _bdot(p, pw2_ref[...]) + pb2_ref[...]


def _head(h_flat, fw1, fb1, fw2, fb2, pw1, pb1, pw2, pb2):
    return pl.pallas_call(
        _head_body,
        out_shape=jax.ShapeDtypeStruct((B, 128), jnp.float32),
    )(h_flat, fw1.T, fb1.reshape(1, -1), fw2.T, fb2.reshape(1, -1),
      pw1.T, pb1.reshape(1, -1), pw2.T, pb2.reshape(1, -1))


NW = 32
SEGS = (B * N) // NW
PAIRS = SEGS // 2


def _sc_edge_kernel(Cp):
    mesh = plsc.VectorSubcoreMesh(core_axis_name="c", subcore_axis_name="s")

    @functools.partial(
        pl.kernel, mesh=mesh,
        compiler_params=pltpu.CompilerParams(use_tc_tiling_on_sc=False),
        out_type=jax.ShapeDtypeStruct((B * K * N, Cp), jnp.float32),
        scratch_types=[
            pltpu.VMEM((PAIRS, 48), jnp.int32),
            pltpu.VMEM((PAIRS, 40), jnp.int32),
            pltpu.VMEM((48, Cp), jnp.float32),
            pltpu.VMEM((40, Cp), jnp.float32),
            pltpu.SemaphoreType.DMA,
            pltpu.SemaphoreType.DMA,
        ],
    )
    def body(tab_hbm, gidx_hbm, sidx_hbm, e3_hbm, gi_v, si_v, rows_v, e_v,
             sem_g, sem_s):
        w = lax.axis_index("s") * 2 + lax.axis_index("c")
        pltpu.sync_copy(gidx_hbm.at[w], gi_v)
        pltpu.sync_copy(sidx_hbm.at[w], si_v)

        def pair(p, carry):
            pltpu.async_copy(tab_hbm.at[gi_v.at[p]], rows_v, sem_g).wait()

            def chunk(ci, c2):
                off = ci * 16
                s0 = rows_v[40, pl.ds(off, 16)]
                s1 = rows_v[41, pl.ds(off, 16)]
                for j in range(K):
                    e_v[j, pl.ds(off, 16)] = rows_v[j, pl.ds(off, 16)] - s0
                    e_v[K + j, pl.ds(off, 16)] = (
                        rows_v[K + j, pl.ds(off, 16)] - s1)
                return c2

            lax.fori_loop(0, Cp // 16, chunk, 0, unroll=False)
            pltpu.async_copy(e_v, e3_hbm.at[si_v.at[p]], sem_s).wait()
            return carry

        lax.fori_loop(0, PAIRS, pair, 0, unroll=False)

    return body


def _edge_features(xt, idx):
    bb, n, Cp = xt.shape
    S = bb * n
    tab = xt.reshape(S, Cp)
    goff = (jnp.arange(bb, dtype=jnp.int32) * n)[:, None, None]
    g = jnp.transpose(idx + goff, (0, 2, 1)).reshape(S, K)
    selfs = jnp.arange(S, dtype=jnp.int32).reshape(S // 2, 2)
    gidx = jnp.concatenate(
        [g.reshape(S // 2, 2 * K), selfs,
         jnp.broadcast_to(selfs[:, :1], (S // 2, 6))], axis=1)
    gidx = gidx.reshape(NW, PAIRS, 48)
    s_all = jnp.arange(S, dtype=jnp.int32)
    b_all = s_all // n
    n_all = s_all % n
    dst = (b_all[:, None] * K + jnp.arange(K, dtype=jnp.int32)[None, :]) * n \
        + n_all[:, None]
    sidx = dst.reshape(NW, PAIRS, 40)
    E = _sc_edge_kernel(Cp)(tab, gidx, sidx)
    return E.reshape(bb, K, n, Cp)


def _layer(xt, W, b):
    bb, n, C = xt.shape
    O = W.shape[0]
    Cp = xt.shape[2]
    idx = _knn(xt)
    E3 = _edge_features(xt, idx)
    waT = W[:, :C].T
    wbT = W[:, C:].T
    M, T1, T2 = _conv_reduce(E3, xt, waT, wbT, b)
    h = _bn_act(M, T1, T2)
    return h.reshape(bb, n, O)


@jax.jit
def kernel(x, W1, b1, g1, be1, W2, b2, g2, be2, W3, b3, g3, be3, W4, b4, g4,
           be4, fw1, fb1, fw2, fb2, pw1, pb1, pw2, pb2):
    xt = jnp.transpose(x, (0, 2, 1))
    xt = jnp.concatenate(
        [xt, jnp.zeros((B, N, 13), jnp.float32)], axis=2)

    h = _layer1(xt, W1, b1)
    h = _layer(h, W2, b2)
    h = _layer(h, W3, b3)
    h = _layer(h, W4, b4)
    return _head(h.reshape(B * N, -1), fw1, fb1, fw2, fb2, pw1, pb1, pw2, pb2)


def _layer1(xtp, W, b):
    bb, n, Cp = xtp.shape
    C = 3
    O = W.shape[0]
    idx = _knn(xtp)
    E3 = _edge_features(xtp, idx)
    z = jnp.zeros((Cp - C, O), jnp.float32)
    waT = jnp.concatenate([W[:, :C].T, z], axis=0)
    wbT = jnp.concatenate([W[:, C:].T, z], axis=0)
    M, T1, T2 = _conv_reduce(E3, xtp, waT, wbT, b)
    h = _bn_act(M, T1, T2)
    return h.reshape(bb, n, O)


# speedup vs baseline: 6.2352x; 1.1285x over previous
import functools
import jax
import jax.numpy as jnp
from jax import lax
from jax.experimental import pallas as pl
from jax.experimental.pallas import tpu as pltpu
from jax.experimental.pallas import tpu_sc as plsc

K = 20
N = 1024
B = 2
NEG = -3.0e38


def _leaky(x):
    return jnp.where(x >= 0, x, 0.01 * x)


def _bdot(a, b):
    return jax.lax.dot_general(a.astype(jnp.bfloat16), b.astype(jnp.bfloat16),
                               (((1,), (0,)), ((), ())),
                               preferred_element_type=jnp.float32)


def _knn_body(x_ref, idx_ref):
    xb = x_ref[0]
    xb16 = xb.astype(jnp.bfloat16)
    G = jax.lax.dot_general(xb16, xb16, (((1,), (1,)), ((), ())),
                            preferred_element_type=jnp.float32)
    xx = jnp.sum(xb * xb, axis=1, keepdims=True)
    D = (-xx - (-2.0 * G)) - xx.reshape(1, N)
    col = jax.lax.broadcasted_iota(jnp.int32, (N, N), 1)
    for t in range(K):
        m = jnp.max(D, axis=1, keepdims=True)
        c = jnp.min(jnp.where(D >= m, col, N), axis=1)
        idx_ref[0, t, :] = c
        D = jnp.where(col == c[:, None], NEG, D)


def _knn(xt):
    b, n, C = xt.shape
    return pl.pallas_call(
        _knn_body,
        grid=(b,),
        in_specs=[pl.BlockSpec((1, N, C), lambda i: (i, 0, 0))],
        out_specs=pl.BlockSpec((1, K, N), lambda i: (i, 0, 0)),
        out_shape=jax.ShapeDtypeStruct((b, K, N), jnp.int32),
    )(xt)


def _conv_body(e_ref, x_ref, wa_ref, wb_ref, b_ref, m_ref, t1_ref, t2_ref):
    blkn = x_ref.shape[1]
    Cp = x_ref.shape[2]
    O = wa_ref.shape[1]
    e = e_ref[0].reshape(K * blkn, Cp)
    y1 = _bdot(e, wa_ref[...])
    y2 = _bdot(x_ref[0], wb_ref[...]) + b_ref[...]
    y = y1.reshape(K, blkn, O) + y2[None, :, :]
    m_ref[0] = jnp.max(y, axis=0)
    t1p = jnp.sum(y, axis=(0, 1), keepdims=False).reshape(1, O)
    t2p = jnp.sum(y * y, axis=(0, 1), keepdims=False).reshape(1, O)

    @pl.when((pl.program_id(0) == 0) & (pl.program_id(1) == 0))
    def _():
        t1_ref[...] = jnp.zeros_like(t1_ref)
        t2_ref[...] = jnp.zeros_like(t2_ref)

    t1_ref[...] += t1p
    t2_ref[...] += t2p


def _conv_reduce(E3, xt, waT, wbT, bias, blkn=128):
    b, n, Cp = xt.shape
    O = waT.shape[1]
    nb = n // blkn
    return pl.pallas_call(
        _conv_body,
        grid=(b, nb),
        in_specs=[
            pl.BlockSpec((1, K, blkn, Cp), lambda i, j: (i, 0, j, 0)),
            pl.BlockSpec((1, blkn, Cp), lambda i, j: (i, j, 0)),
            pl.BlockSpec((Cp, O), lambda i, j: (0, 0)),
            pl.BlockSpec((Cp, O), lambda i, j: (0, 0)),
            pl.BlockSpec((1, O), lambda i, j: (0, 0)),
        ],
        out_specs=[
            pl.BlockSpec((1, blkn, O), lambda i, j: (i, j, 0)),
            pl.BlockSpec((1, O), lambda i, j: (0, 0)),
            pl.BlockSpec((1, O), lambda i, j: (0, 0)),
        ],
        out_shape=[
            jax.ShapeDtypeStruct((b, n, O), jnp.float32),
            jax.ShapeDtypeStruct((1, O), jnp.float32),
            jax.ShapeDtypeStruct((1, O), jnp.float32),
        ],
    )(E3, xt, waT, wbT, bias.reshape(1, O))


def _bn_body(m_ref, t1_ref, t2_ref, h_ref):
    cnt = float(B * N * K)
    mean = t1_ref[...] / cnt
    var = t2_ref[...] / cnt - mean * mean
    h = (m_ref[...] - mean) / jnp.sqrt(var + 1e-5)
    h_ref[...] = _leaky(h)


def _bn_act(M, T1, T2):
    b, n, O = M.shape
    return pl.pallas_call(
        _bn_body,
        out_shape=jax.ShapeDtypeStruct((b * n, O), jnp.float32),
    )(M.reshape(b * n, O), T1, T2)


def _head_body(h_ref, fw1_ref, fb1_ref, fw2_ref, fb2_ref, pw1_ref, pb1_ref,
               pw2_ref, pb2_ref, o_ref):
    h = h_ref[...]
    parts = []
    for i in range(B):
        hb = h[i * N:(i + 1) * N]
        parts.append(jnp.concatenate(
            [jnp.mean(hb, axis=0, keepdims=True),
             jnp.max(hb, axis=0, keepdims=True)], axis=1))
    z = jnp.concatenate(parts, axis=0)
    e = _leaky(_bdot(z, fw1_ref[...]) + fb1_ref[...])
    e = _bdot(e, fw2_ref[...]) + fb2_ref[...]
    p = _leaky(_bdot(e, pw1_ref[...]) + pb1_ref[...])
    o_ref[...] = _bdot(p, pw2_ref[...]) + pb2_ref[...]


def _head(h_flat, fw1, fb1, fw2, fb2, pw1, pb1, pw2, pb2):
    return pl.pallas_call(
        _head_body,
        out_shape=jax.ShapeDtypeStruct((B, 128), jnp.float32),
    )(h_flat, fw1.T, fb1.reshape(1, -1), fw2.T, fb2.reshape(1, -1),
      pw1.T, pb1.reshape(1, -1), pw2.T, pb2.reshape(1, -1))


NW = 32
SEGS = (B * N) // NW
PAIRS = SEGS // 2


def _sc_edge_kernel(Cp):
    mesh = plsc.VectorSubcoreMesh(core_axis_name="c", subcore_axis_name="s")

    @functools.partial(
        pl.kernel, mesh=mesh,
        compiler_params=pltpu.CompilerParams(use_tc_tiling_on_sc=False),
        out_type=jax.ShapeDtypeStruct((B * K * N, Cp), jnp.float32),
        scratch_types=[
            pltpu.VMEM((PAIRS, 48), jnp.int32),
            pltpu.VMEM((PAIRS, 40), jnp.int32),
            pltpu.VMEM((2, 48, Cp), jnp.float32),
            pltpu.VMEM((2, 40, Cp), jnp.float32),
            pltpu.SemaphoreType.DMA,
            pltpu.SemaphoreType.DMA,
            pltpu.SemaphoreType.DMA,
            pltpu.SemaphoreType.DMA,
        ],
    )
    def body(tab_hbm, gidx_hbm, sidx_hbm, e3_hbm, gi_v, si_v, rows_v, e_v,
             sem_g0, sem_g1, sem_s0, sem_s1):
        w = lax.axis_index("s") * 2 + lax.axis_index("c")
        pltpu.sync_copy(gidx_hbm.at[w], gi_v)
        pltpu.sync_copy(sidx_hbm.at[w], si_v)

        def compute(buf):
            def chunk(ci, c2):
                off = ci * 16
                s0 = rows_v[buf, 40, pl.ds(off, 16)]
                s1 = rows_v[buf, 41, pl.ds(off, 16)]
                for j in range(K):
                    e_v[buf, j, pl.ds(off, 16)] = (
                        rows_v[buf, j, pl.ds(off, 16)] - s0)
                    e_v[buf, K + j, pl.ds(off, 16)] = (
                        rows_v[buf, K + j, pl.ds(off, 16)] - s1)
                return c2

            lax.fori_loop(0, Cp // 16, chunk, 0, unroll=False)

        pltpu.async_copy(tab_hbm.at[gi_v.at[0]], rows_v.at[0], sem_g0)

        def step(i, carry):
            pa = 2 * i
            pb = pa + 1
            pltpu.async_copy(tab_hbm.at[gi_v.at[pb]], rows_v.at[1], sem_g1)
            pltpu.make_async_copy(
                tab_hbm.at[gi_v.at[pa]], rows_v.at[0], sem_g0).wait()

            @pl.when(i > 0)
            def _():
                pltpu.make_async_copy(
                    e_v.at[0], e3_hbm.at[si_v.at[pa]], sem_s0).wait()

            compute(0)
            pltpu.async_copy(e_v.at[0], e3_hbm.at[si_v.at[pa]], sem_s0)
            pnx = jnp.minimum(pa + 2, PAIRS - 1)
            pltpu.async_copy(tab_hbm.at[gi_v.at[pnx]], rows_v.at[0], sem_g0)

            pltpu.make_async_copy(
                tab_hbm.at[gi_v.at[pb]], rows_v.at[1], sem_g1).wait()

            @pl.when(i > 0)
            def _():
                pltpu.make_async_copy(
                    e_v.at[1], e3_hbm.at[si_v.at[pb]], sem_s1).wait()

            compute(1)
            pltpu.async_copy(e_v.at[1], e3_hbm.at[si_v.at[pb]], sem_s1)
            return carry

        lax.fori_loop(0, PAIRS // 2, step, 0, unroll=False)
        pltpu.make_async_copy(
            tab_hbm.at[gi_v.at[PAIRS - 1]], rows_v.at[0], sem_g0).wait()
        pltpu.make_async_copy(
            e_v.at[0], e3_hbm.at[si_v.at[PAIRS - 2]], sem_s0).wait()
        pltpu.make_async_copy(
            e_v.at[1], e3_hbm.at[si_v.at[PAIRS - 1]], sem_s1).wait()

    return body


def _edge_features(xt, idx):
    bb, n, Cp = xt.shape
    S = bb * n
    tab = xt.reshape(S, Cp)
    goff = (jnp.arange(bb, dtype=jnp.int32) * n)[:, None, None]
    g = jnp.transpose(idx + goff, (0, 2, 1)).reshape(S, K)
    selfs = jnp.arange(S, dtype=jnp.int32).reshape(S // 2, 2)
    gidx = jnp.concatenate(
        [g.reshape(S // 2, 2 * K), selfs,
         jnp.broadcast_to(selfs[:, :1], (S // 2, 6))], axis=1)
    gidx = gidx.reshape(NW, PAIRS, 48)
    s_all = jnp.arange(S, dtype=jnp.int32)
    b_all = s_all // n
    n_all = s_all % n
    dst = (b_all[:, None] * K + jnp.arange(K, dtype=jnp.int32)[None, :]) * n \
        + n_all[:, None]
    sidx = dst.reshape(NW, PAIRS, 40)
    E = _sc_edge_kernel(Cp)(tab, gidx, sidx)
    return E.reshape(bb, K, n, Cp)


def _layer(xt, W, b):
    bb, n, C = xt.shape
    O = W.shape[0]
    Cp = xt.shape[2]
    idx = _knn(xt)
    E3 = _edge_features(xt, idx)
    waT = W[:, :C].T
    wbT = W[:, C:].T
    M, T1, T2 = _conv_reduce(E3, xt, waT, wbT, b)
    h = _bn_act(M, T1, T2)
    return h.reshape(bb, n, O)


@jax.jit
def kernel(x, W1, b1, g1, be1, W2, b2, g2, be2, W3, b3, g3, be3, W4, b4, g4,
           be4, fw1, fb1, fw2, fb2, pw1, pb1, pw2, pb2):
    xt = jnp.transpose(x, (0, 2, 1))
    xt = jnp.concatenate(
        [xt, jnp.zeros((B, N, 13), jnp.float32)], axis=2)

    h = _layer1(xt, W1, b1)
    h = _layer(h, W2, b2)
    h = _layer(h, W3, b3)
    h = _layer(h, W4, b4)
    return _head(h.reshape(B * N, -1), fw1, fb1, fw2, fb2, pw1, pb1, pw2, pb2)


def _layer1(xtp, W, b):
    bb, n, Cp = xtp.shape
    C = 3
    O = W.shape[0]
    idx = _knn(xtp)
    E3 = _edge_features(xtp, idx)
    z = jnp.zeros((Cp - C, O), jnp.float32)
    waT = jnp.concatenate([W[:, :C].T, z], axis=0)
    wbT = jnp.concatenate([W[:, C:].T, z], axis=0)
    M, T1, T2 = _conv_reduce(E3, xtp, waT, wbT, b)
    h = _bn_act(M, T1, T2)
    return h.reshape(bb, n, O)
